# baseline (speedup 1.0000x reference)
import jax
import jax.numpy as jnp
from jax import lax
from jax.experimental import pallas as pl
from jax.experimental.pallas import tpu as pltpu, tpu_sc as plsc

N_NODES = 10000
N_EDGES = 320000
D_FEAT = 128
HIDDEN = 64
N_CLASSES = 40
CPAD = 48

NC = 2
NS = 16
NW = NC * NS
K = 256
CH = (N_EDGES + NW * K - 1) // (NW * K)
EPAD = NW * CH * K
NPAD = ((N_NODES + 1) + NS * 8 - 1) // (NS * 8) * (NS * 8)
RPS = NPAD // NS

def _mesh():
    return plsc.VectorSubcoreMesh(
        core_axis_name="c", subcore_axis_name="s", num_cores=NC, num_subcores=NS)


def _deg_body(dst_hbm, ones_hbm, zeros_hbm, out_hbm, idx_v, ones_v, acc_sh):
    c = lax.axis_index("c")
    s = lax.axis_index("s")
    wid = s * NC + c
    rs = pl.ds(s * RPS, RPS)
    pltpu.sync_copy(zeros_hbm.at[rs], acc_sh.at[rs])
    pltpu.sync_copy(ones_hbm, ones_v)
    pltpu.sync_copy(dst_hbm.at[wid], idx_v)
    plsc.subcore_barrier()
    for j in range(CH):
        pltpu.sync_copy(ones_v, acc_sh.at[idx_v.at[j]], add=True)
    plsc.subcore_barrier()
    pltpu.sync_copy(acc_sh.at[rs], out_hbm.at[c].at[rs])


def _make_deg():
    return pl.kernel(
        _deg_body,
        out_type=jax.ShapeDtypeStruct((NC, NPAD, 16), jnp.float32),
        mesh=_mesh(),
        scratch_types=[
            pltpu.VMEM((CH, K), jnp.int32),
            pltpu.VMEM((K, 16), jnp.float32),
            pltpu.VMEM_SHARED((NPAD, 16), jnp.float32),
        ],
        compiler_params=pltpu.CompilerParams(use_tc_tiling_on_sc=False),
    )


NB = 4


def _make_scatter(width):
    def body(y_hbm, src_hbm, dst_hbm, zeros_hbm, out_hbm,
             src_v, dst_v, bufs, acc_sh, *sems):
        c = lax.axis_index("c")
        s = lax.axis_index("s")
        wid = s * NC + c
        rs = pl.ds(s * RPS, RPS)
        pltpu.sync_copy(zeros_hbm.at[rs], acc_sh.at[rs])
        pltpu.sync_copy(src_hbm.at[wid], src_v)
        pltpu.sync_copy(dst_hbm.at[wid], dst_v)
        plsc.subcore_barrier()
        gsems, ssems = sems[:NB], sems[NB:]
        gdesc = [None] * CH
        sdesc = [None] * CH

        def gather(j):
            return pltpu.async_copy(
                y_hbm.at[src_v.at[j]], bufs.at[j % NB], gsems[j % NB])

        for j in range(min(2, CH)):
            gdesc[j] = gather(j)
        for j in range(CH):
            if j + 2 < CH:
                if j + 2 - NB >= 0:
                    sdesc[j + 2 - NB].wait()
                gdesc[j + 2] = gather(j + 2)
            gdesc[j].wait()
            sdesc[j] = pltpu.async_copy(
                bufs.at[j % NB], acc_sh.at[dst_v.at[j]], ssems[j % NB],
                add=True)
        for j in range(max(0, CH - NB), CH):
            sdesc[j].wait()
        plsc.subcore_barrier()
        pltpu.sync_copy(acc_sh.at[rs], out_hbm.at[c].at[rs])

    return pl.kernel(
        body,
        out_type=jax.ShapeDtypeStruct((NC, NPAD, width), jnp.float32),
        mesh=_mesh(),
        scratch_types=[
            pltpu.VMEM((CH, K), jnp.int32),
            pltpu.VMEM((CH, K), jnp.int32),
            pltpu.VMEM((NB, K, width), jnp.float32),
            pltpu.VMEM_SHARED((NPAD, width), jnp.float32),
        ] + [pltpu.SemaphoreType.DMA] * (2 * NB),
        compiler_params=pltpu.CompilerParams(use_tc_tiling_on_sc=False),
    )


def _tc1_body(dega_ref, x_ref, w1_ref, y1_ref, dinv_ref):
    deg = dega_ref[0, :N_NODES, 0] + dega_ref[1, :N_NODES, 0] + 1.0
    dinv = jnp.where(deg > 0, 1.0 / jnp.sqrt(deg), 0.0)
    dinv_ref[...] = dinv
    xw = jnp.dot(x_ref[...], w1_ref[...], preferred_element_type=jnp.float32)
    y1_ref[...] = xw * dinv[:, None]


def _tc2_body(z_ref, y1_ref, dinv_ref, b1_ref, w2_ref, y2_ref):
    dinv = dinv_ref[...]
    z = z_ref[0, :N_NODES, :] + z_ref[1, :N_NODES, :] + y1_ref[...]
    h = jnp.maximum(z * dinv[:, None] + b1_ref[...][None, :], 0.0)
    y2_ref[...] = jnp.dot(
        h, w2_ref[...], preferred_element_type=jnp.float32) * dinv[:, None]


def _tc3_body(z_ref, y2_ref, dinv_ref, b2_ref, out_ref):
    o = (z_ref[0, :N_NODES, :N_CLASSES] + z_ref[1, :N_NODES, :N_CLASSES]
         + y2_ref[:, :N_CLASSES]) * dinv_ref[...][:, None] + b2_ref[...][None, :]
    m = jnp.max(o, axis=1, keepdims=True)
    lse = jnp.log(jnp.sum(jnp.exp(o - m), axis=1, keepdims=True))
    out_ref[...] = o - m - lse


def kernel(x, edge_index, W1, b1, W2, b2):
    ei = edge_index.astype(jnp.int32)
    pad = EPAD - N_EDGES
    src_p = jnp.concatenate(
        [ei[0], jnp.zeros((pad,), jnp.int32)]).reshape(NW, CH, K)
    dst_p = jnp.concatenate(
        [ei[1], jnp.full((pad,), N_NODES, jnp.int32)]).reshape(NW, CH, K)
    ones16 = jnp.ones((K, 16), jnp.float32)
    zeros16 = jnp.zeros((NPAD, 16), jnp.float32)
    zeros_h = jnp.zeros((NPAD, HIDDEN), jnp.float32)
    zeros_c = jnp.zeros((NPAD, CPAD), jnp.float32)
    w2p = jnp.pad(W2, ((0, 0), (0, CPAD - N_CLASSES)))

    dega = _make_deg()(dst_p, ones16, zeros16)

    y1, dinv = pl.pallas_call(
        _tc1_body,
        out_shape=[
            jax.ShapeDtypeStruct((N_NODES, HIDDEN), jnp.float32),
            jax.ShapeDtypeStruct((N_NODES,), jnp.float32),
        ],
    )(dega, x, W1)

    z1 = _make_scatter(HIDDEN)(y1, src_p, dst_p, zeros_h)

    y2 = pl.pallas_call(
        _tc2_body,
        out_shape=jax.ShapeDtypeStruct((N_NODES, CPAD), jnp.float32),
    )(z1, y1, dinv, b1, w2p)

    z2 = _make_scatter(CPAD)(y2, src_p, dst_p, zeros_c)

    out = pl.pallas_call(
        _tc3_body,
        out_shape=jax.ShapeDtypeStruct((N_NODES, N_CLASSES), jnp.float32),
    )(z2, y2, dinv, b2)
    return out


# speedup vs baseline: 1.8334x; 1.8334x over previous
import jax
import jax.numpy as jnp
from jax import lax
from jax.experimental import pallas as pl
from jax.experimental.pallas import tpu as pltpu, tpu_sc as plsc

N_NODES = 10000
N_EDGES = 320000
D_FEAT = 128
HIDDEN = 64
N_CLASSES = 40
CPAD = 48

NC = 2
NS = 16
NW = NC * NS
K = 64
CH = (N_EDGES + NW * K - 1) // (NW * K)
EPAD = NW * CH * K
NPAD = ((N_NODES + 1) + NS * 8 - 1) // (NS * 8) * (NS * 8)
RPS = NPAD // NS

def _mesh():
    return plsc.VectorSubcoreMesh(
        core_axis_name="c", subcore_axis_name="s", num_cores=NC, num_subcores=NS)


def _deg_body(dst_hbm, ones_hbm, zeros_hbm, out_hbm, idx_v, ones_v, acc_sh):
    c = lax.axis_index("c")
    s = lax.axis_index("s")
    wid = s * NC + c
    rs = pl.ds(s * RPS, RPS)
    pltpu.sync_copy(zeros_hbm.at[rs], acc_sh.at[rs])
    pltpu.sync_copy(ones_hbm, ones_v)
    pltpu.sync_copy(dst_hbm.at[wid], idx_v)
    plsc.subcore_barrier()
    for j in range(CH):
        pltpu.sync_copy(ones_v, acc_sh.at[idx_v.at[j]], add=True)
    plsc.subcore_barrier()
    pltpu.sync_copy(acc_sh.at[rs], out_hbm.at[c].at[rs])


def _make_deg():
    return pl.kernel(
        _deg_body,
        out_type=jax.ShapeDtypeStruct((NC, NPAD, 16), jnp.float32),
        mesh=_mesh(),
        scratch_types=[
            pltpu.VMEM((CH, K), jnp.int32),
            pltpu.VMEM((K, 16), jnp.float32),
            pltpu.VMEM_SHARED((NPAD, 16), jnp.float32),
        ],
        compiler_params=pltpu.CompilerParams(use_tc_tiling_on_sc=False),
    )


NB = 4


def _make_scatter(width):
    def body(y_hbm, src_hbm, dst_hbm, zeros_hbm, out_hbm,
             src_v, dst_v, bufs, acc_sh, *sems):
        c = lax.axis_index("c")
        s = lax.axis_index("s")
        wid = s * NC + c
        rs = pl.ds(s * RPS, RPS)
        pltpu.sync_copy(zeros_hbm.at[rs], acc_sh.at[rs])
        pltpu.sync_copy(src_hbm.at[wid], src_v)
        pltpu.sync_copy(dst_hbm.at[wid], dst_v)
        plsc.subcore_barrier()
        gsems, ssems = sems[:NB], sems[NB:]
        gdesc = [None] * CH
        sdesc = [None] * CH

        def gather(j):
            return pltpu.async_copy(
                y_hbm.at[src_v.at[j]], bufs.at[j % NB], gsems[j % NB])

        for j in range(min(2, CH)):
            gdesc[j] = gather(j)
        for j in range(CH):
            if j + 2 < CH:
                if j + 2 - NB >= 0:
                    sdesc[j + 2 - NB].wait()
                gdesc[j + 2] = gather(j + 2)
            gdesc[j].wait()
            sdesc[j] = pltpu.async_copy(
                bufs.at[j % NB], acc_sh.at[dst_v.at[j]], ssems[j % NB],
                add=True)
        for j in range(max(0, CH - NB), CH):
            sdesc[j].wait()
        plsc.subcore_barrier()
        pltpu.sync_copy(acc_sh.at[rs], out_hbm.at[c].at[rs])

    return pl.kernel(
        body,
        out_type=jax.ShapeDtypeStruct((NC, NPAD, width), jnp.float32),
        mesh=_mesh(),
        scratch_types=[
            pltpu.VMEM((CH, K), jnp.int32),
            pltpu.VMEM((CH, K), jnp.int32),
            pltpu.VMEM((NB, K, width), jnp.float32),
            pltpu.VMEM_SHARED((NPAD, width), jnp.float32),
        ] + [pltpu.SemaphoreType.DMA] * (2 * NB),
        compiler_params=pltpu.CompilerParams(use_tc_tiling_on_sc=False),
    )


def _tc1_body(dega_ref, x_ref, w1_ref, y1_ref, dinv_ref):
    deg = dega_ref[0, :N_NODES, 0] + dega_ref[1, :N_NODES, 0] + 1.0
    dinv = jnp.where(deg > 0, 1.0 / jnp.sqrt(deg), 0.0)
    dinv_ref[...] = dinv
    xw = jnp.dot(x_ref[...], w1_ref[...], preferred_element_type=jnp.float32)
    y1_ref[...] = xw * dinv[:, None]


def _tc2_body(z_ref, y1_ref, dinv_ref, b1_ref, w2_ref, y2_ref):
    dinv = dinv_ref[...]
    z = z_ref[0, :N_NODES, :] + z_ref[1, :N_NODES, :] + y1_ref[...]
    h = jnp.maximum(z * dinv[:, None] + b1_ref[...][None, :], 0.0)
    y2_ref[...] = jnp.dot(
        h, w2_ref[...], preferred_element_type=jnp.float32) * dinv[:, None]


def _tc3_body(z_ref, y2_ref, dinv_ref, b2_ref, out_ref):
    o = (z_ref[0, :N_NODES, :N_CLASSES] + z_ref[1, :N_NODES, :N_CLASSES]
         + y2_ref[:, :N_CLASSES]) * dinv_ref[...][:, None] + b2_ref[...][None, :]
    m = jnp.max(o, axis=1, keepdims=True)
    lse = jnp.log(jnp.sum(jnp.exp(o - m), axis=1, keepdims=True))
    out_ref[...] = o - m - lse


def kernel(x, edge_index, W1, b1, W2, b2):
    ei = edge_index.astype(jnp.int32)
    pad = EPAD - N_EDGES
    src_p = jnp.concatenate(
        [ei[0], jnp.zeros((pad,), jnp.int32)]).reshape(NW, CH, K)
    dst_p = jnp.concatenate(
        [ei[1], jnp.full((pad,), N_NODES, jnp.int32)]).reshape(NW, CH, K)
    ones16 = jnp.ones((K, 16), jnp.float32)
    zeros16 = jnp.zeros((NPAD, 16), jnp.float32)
    zeros_h = jnp.zeros((NPAD, HIDDEN), jnp.float32)
    zeros_c = jnp.zeros((NPAD, CPAD), jnp.float32)
    w2p = jnp.pad(W2, ((0, 0), (0, CPAD - N_CLASSES)))

    dega = _make_deg()(dst_p, ones16, zeros16)

    y1, dinv = pl.pallas_call(
        _tc1_body,
        out_shape=[
            jax.ShapeDtypeStruct((N_NODES, HIDDEN), jnp.float32),
            jax.ShapeDtypeStruct((N_NODES,), jnp.float32),
        ],
    )(dega, x, W1)

    z1 = _make_scatter(HIDDEN)(y1, src_p, dst_p, zeros_h)

    y2 = pl.pallas_call(
        _tc2_body,
        out_shape=jax.ShapeDtypeStruct((N_NODES, CPAD), jnp.float32),
    )(z1, y1, dinv, b1, w2p)

    z2 = _make_scatter(CPAD)(y2, src_p, dst_p, zeros_c)

    out = pl.pallas_call(
        _tc3_body,
        out_shape=jax.ShapeDtypeStruct((N_NODES, N_CLASSES), jnp.float32),
    )(z2, y2, dinv, b2)
    return out
